# baseline (speedup 1.0000x reference)
import functools

import jax
import jax.numpy as jnp
from jax import lax
from jax.experimental import pallas as pl
from jax.experimental.pallas import tpu as pltpu
from jax.experimental.pallas import tpu_sc as plsc

N = 10000
E = 320000
R = 8
C = 150
CP = 160
K = R * CP
H = 128
NEV = 49

NC = 2
NS = 16
ET = E // NS
STAGE = 2000
NSTAGE = ET // STAGE
VPC = STAGE // 16
NODES_PER_CORE = N // NC
PASSES = 5
NP_ = NODES_PER_CORE // PASSES
SW = NP_ * K
SLAB = SW + 128
TSHARE = SW // NS
ZBUF = 2000
NZ = TSHARE // ZBUF
CHUNK = 128
RING = 8
CLCAP = ET + 1152


def _sc_body(edge_hbm, rel_hbm, x_hbm, hist_hbm,
             slab, xbuf, keys, srcb, dstb, relb, idxb, ones, zbuf, bbuf,
             bbuf2, sem, sem2, sem3):
    c = lax.axis_index("c")
    s = lax.axis_index("s")
    tile_base = s * ET

    pltpu.sync_copy(x_hbm, xbuf)

    def fill_ones(i, carry):
        ones[pl.ds(i * 16, 16)] = jnp.ones((16,), jnp.float32)
        return carry
    lax.fori_loop(0, 8, fill_ones, 0)

    def fill_zeros(i, carry):
        zbuf[pl.ds(i * 16, 16)] = jnp.zeros((16,), jnp.float32)
        return carry
    lax.fori_loop(0, ZBUF // 16, fill_zeros, 0)

    lo_core = c * NODES_PER_CORE

    def stage_chunk(j, pos):
        off = tile_base + j * STAGE
        pltpu.sync_copy(edge_hbm.at[pl.ds(off, STAGE)], srcb)
        pltpu.sync_copy(edge_hbm.at[pl.ds(E + off, STAGE)], dstb)
        pltpu.sync_copy(rel_hbm.at[pl.ds(off, STAGE)], relb)

        def keyvec(v, pos2):
            sv = srcb[pl.ds(v * 16, 16)]
            dv = dstb[pl.ds(v * 16, 16)]
            rv = relb[pl.ds(v * 16, 16)]
            cls = plsc.load_gather(xbuf, [sv])
            kv = dv * K + rv * CP + cls
            valid = (dv >= lo_core) & (dv < lo_core + NODES_PER_CORE)
            plsc.store_compressed(keys.at[pl.ds(pos2, 16)], kv, mask=valid)
            return pos2 + jnp.sum(valid.astype(jnp.int32))
        return lax.fori_loop(0, VPC, keyvec, pos)
    pos = lax.fori_loop(0, NSTAGE, stage_chunk, jnp.int32(0))

    def fill_tail(t, carry):
        keys[pl.ds(pos + t * 16, 16)] = jnp.full((16,), -1, jnp.int32)
        return carry
    lax.fori_loop(0, 66, fill_tail, 0)
    ngr = (pos + jnp.int32(RING * CHUNK - 1)) // (RING * CHUNK)

    trash = jnp.int32(SW) + s * 4

    for p in range(PASSES):
        lo = c * (NODES_PER_CORE * K) + jnp.int32(p * NP_ * K)

        if p == 0:
            zdescs = [
                pltpu.async_copy(
                    zbuf, slab.at[pl.ds(s * TSHARE + z * ZBUF, ZBUF)], sem)
                for z in range(NZ)
            ]
            for dsc in zdescs:
                dsc.wait()
            plsc.subcore_barrier()

        def scatter_group(g, carry):
            descs = []
            for b in range(RING):
                base = g * (RING * CHUNK) + b * CHUNK
                for v in range(CHUNK // 16):
                    kv = keys[pl.ds(base + v * 16, 16)]
                    off = kv - lo
                    valid = (off >= 0) & (off < SW)
                    idxb[b, pl.ds(v * 16, 16)] = jnp.where(valid, off, trash)
                descs.append(
                    pltpu.async_copy(ones, slab.at[idxb.at[b]], sem, add=True))
            for d in descs:
                d.wait()
            return carry
        lax.fori_loop(0, ngr, scatter_group, 0)
        plsc.subcore_barrier()

        hbase = c * (NODES_PER_CORE * K) + jnp.int32(p * NP_ * K) + s * TSHARE
        bb = (bbuf, bbuf2)
        ind = [None, None]
        outd = [None, None]
        ind[0] = pltpu.async_copy(
            slab.at[pl.ds(s * TSHARE, ZBUF)], bb[0], sem)
        zdescs = []
        for z in range(NZ):
            b = z % 2
            ind[b].wait()
            if z >= 2:
                outd[b].wait()
            outd[b] = pltpu.async_copy(
                bb[b], hist_hbm.at[pl.ds(hbase + z * ZBUF, ZBUF)], sem2)
            if z + 1 < NZ:
                ind[1 - b] = pltpu.async_copy(
                    slab.at[pl.ds(s * TSHARE + (z + 1) * ZBUF, ZBUF)],
                    bb[1 - b], sem)
            if p + 1 < PASSES:
                zdescs.append(pltpu.async_copy(
                    zbuf, slab.at[pl.ds(s * TSHARE + z * ZBUF, ZBUF)], sem3))
        outd[0].wait()
        outd[1].wait()
        for dsc in zdescs:
            dsc.wait()
        plsc.subcore_barrier()


@jax.jit
def _sc_hist(edge_index, rel, xflat):
    mesh = plsc.VectorSubcoreMesh(core_axis_name="c", subcore_axis_name="s")
    return pl.kernel(
        _sc_body,
        out_type=jax.ShapeDtypeStruct((N * K,), jnp.float32),
        mesh=mesh,
        compiler_params=pltpu.CompilerParams(needs_layout_passes=False),
        scratch_types=[
            pltpu.VMEM_SHARED((SLAB,), jnp.float32),
            pltpu.VMEM((N,), jnp.int32),
            pltpu.VMEM((CLCAP,), jnp.int32),
            pltpu.VMEM((STAGE,), jnp.int32),
            pltpu.VMEM((STAGE,), jnp.int32),
            pltpu.VMEM((STAGE,), jnp.int32),
            pltpu.VMEM((RING, CHUNK), jnp.int32),
            pltpu.VMEM((CHUNK,), jnp.float32),
            pltpu.VMEM((ZBUF,), jnp.float32),
            pltpu.VMEM((ZBUF,), jnp.float32),
            pltpu.VMEM((ZBUF,), jnp.float32),
            pltpu.SemaphoreType.DMA,
            pltpu.SemaphoreType.DMA,
            pltpu.SemaphoreType.DMA,
        ],
    )(edge_index, rel, xflat)


BN = 2000
GRID = N // BN
TS = (R + 1) * CP
KP = 1536


def _tc_body(hist_ref, nep_ref, wrel_ref, wroot_ref, bias_ref, x_ref,
             nt_ref, evw_ref, evb_ref, out_ref, ts_ref, big_ref, acc_ref,
             cnt_ref):
    step = pl.program_id(0)

    @pl.when(step == 0)
    def _():
        nep = jnp.concatenate(
            [nep_ref[...], jnp.zeros((CP - C, H), jnp.float32)], axis=0)
        for r in range(R):
            ts_ref[r * CP:(r + 1) * CP, :] = jnp.dot(
                nep, wrel_ref[r], preferred_element_type=jnp.float32)
        ts_ref[R * CP:TS, :] = jnp.dot(
            nep, wroot_ref[...], preferred_element_type=jnp.float32)
        acc_ref[...] = jnp.zeros((1, H), jnp.float32)
        cnt_ref[0, 0] = 0.0

    hist = hist_ref[...]
    for r in range(R):
        hr = hist[:, r * CP:(r + 1) * CP]
        cnt = jnp.sum(hr, axis=1, keepdims=True)
        big_ref[:, r * CP:(r + 1) * CP] = hr * (1.0 / jnp.maximum(cnt, 1.0))
    xb = x_ref[...]
    iot = lax.broadcasted_iota(jnp.int32, (BN, CP), 1)
    oh = (iot == xb).astype(jnp.float32)
    root = jnp.dot(oh, ts_ref[R * CP:TS, :],
                   preferred_element_type=jnp.float32)
    agg = jnp.dot(big_ref[...], ts_ref[:K, :],
                  preferred_element_type=jnp.float32)
    h = jnp.maximum(agg + root + bias_ref[...].reshape(1, H), 0.0)
    m = (nt_ref[...] == 0).astype(jnp.float32)
    acc_ref[...] += jnp.sum(h * m, axis=0, keepdims=True)
    cnt_ref[0, 0] += jnp.sum(m)

    @pl.when(step == pl.num_programs(0) - 1)
    def _():
        g = acc_ref[...] / jnp.maximum(cnt_ref[0, 0], 1.0)
        out_ref[...] = lax.dot_general(
            g, evw_ref[...], (((1,), (1,)), ((), ())),
            preferred_element_type=jnp.float32) + evb_ref[...].reshape(1, NEV)


@jax.jit
def _tc_head(hist2d, nep, W_rel, W_root, bias2, xi, nti, ev_W, evb2):
    return pl.pallas_call(
        _tc_body,
        grid=(GRID,),
        in_specs=[
            pl.BlockSpec((BN, K), lambda i: (i, 0)),
            pl.BlockSpec((C, H), lambda i: (0, 0)),
            pl.BlockSpec((R, H, H), lambda i: (0, 0, 0)),
            pl.BlockSpec((H, H), lambda i: (0, 0)),
            pl.BlockSpec((H,), lambda i: (0,)),
            pl.BlockSpec((BN, 1), lambda i: (i, 0)),
            pl.BlockSpec((BN, 1), lambda i: (i, 0)),
            pl.BlockSpec((NEV, H), lambda i: (0, 0)),
            pl.BlockSpec((NEV,), lambda i: (0,)),
        ],
        out_specs=pl.BlockSpec((1, NEV), lambda i: (0, 0)),
        out_shape=jax.ShapeDtypeStruct((1, NEV), jnp.float32),
        scratch_shapes=[
            pltpu.VMEM((TS, H), jnp.float32),
            pltpu.VMEM((BN, K), jnp.float32),
            pltpu.VMEM((1, H), jnp.float32),
            pltpu.SMEM((1, 1), jnp.float32),
        ],
    )(hist2d, nep, W_rel, W_root, bias2, xi, nti, ev_W, evb2)


def kernel(x, edge_index, edge_type, node_type, node_emb, W_rel, W_root,
           bias, ev_W, ev_b):
    xflat = x[:, 0].astype(jnp.int32)
    hist = _sc_hist(edge_index.astype(jnp.int32).reshape(2 * E),
                    edge_type.astype(jnp.int32), xflat).reshape(N, K)

    return _tc_head(hist, node_emb, W_rel, W_root, bias, x.astype(jnp.int32),
                    node_type.reshape(N, 1).astype(jnp.int32), ev_W, ev_b)


# speedup vs baseline: 1.2138x; 1.2138x over previous
import functools

import jax
import jax.numpy as jnp
from jax import lax
from jax.experimental import pallas as pl
from jax.experimental.pallas import tpu as pltpu
from jax.experimental.pallas import tpu_sc as plsc

N = 10000
E = 320000
R = 8
C = 150
CP = 160
K = R * CP
H = 128
NEV = 49

NC = 2
NS = 16
ET = E // NS
STAGE = 2000
NSTAGE = ET // STAGE
VPC = STAGE // 16
NODES_PER_CORE = N // NC
PASSES = 5
NP_ = NODES_PER_CORE // PASSES
SW = NP_ * K
SLAB = SW + 128
TSHARE = SW // NS
ZBUF = 2000
NZ = TSHARE // ZBUF
CHUNK = 128
RING = 8
CLCAP = ET + 1152


def _sc_body(edge_hbm, rel_hbm, x_hbm, hist_hbm,
             slab, xbuf, keys, srcb, dstb, relb, idxb, ones, zbuf, bounce,
             sem, sem2):
    c = lax.axis_index("c")
    s = lax.axis_index("s")
    tile_base = s * ET

    pltpu.sync_copy(x_hbm, xbuf)

    def fill_ones(i, carry):
        ones[pl.ds(i * 16, 16)] = jnp.ones((16,), jnp.float32)
        return carry
    lax.fori_loop(0, 8, fill_ones, 0)

    def fill_zeros(i, carry):
        zbuf[pl.ds(i * 16, 16)] = jnp.zeros((16,), jnp.float32)
        return carry
    lax.fori_loop(0, ZBUF // 16, fill_zeros, 0)

    lo_core = c * NODES_PER_CORE

    def stage_chunk(j, pos):
        off = tile_base + j * STAGE
        pltpu.sync_copy(edge_hbm.at[pl.ds(off, STAGE)], srcb)
        pltpu.sync_copy(edge_hbm.at[pl.ds(E + off, STAGE)], dstb)
        pltpu.sync_copy(rel_hbm.at[pl.ds(off, STAGE)], relb)

        def keyvec(v, pos2):
            sv = srcb[pl.ds(v * 16, 16)]
            dv = dstb[pl.ds(v * 16, 16)]
            rv = relb[pl.ds(v * 16, 16)]
            cls = plsc.load_gather(xbuf, [sv])
            kv = dv * K + rv * CP + cls
            valid = (dv >= lo_core) & (dv < lo_core + NODES_PER_CORE)
            plsc.store_compressed(keys.at[pl.ds(pos2, 16)], kv, mask=valid)
            return pos2 + jnp.sum(valid.astype(jnp.int32))
        return lax.fori_loop(0, VPC, keyvec, pos)
    pos = lax.fori_loop(0, NSTAGE, stage_chunk, jnp.int32(0))

    def fill_tail(t, carry):
        keys[pl.ds(pos + t * 16, 16)] = jnp.full((16,), -1, jnp.int32)
        return carry
    lax.fori_loop(0, 66, fill_tail, 0)
    ngr = (pos + jnp.int32(RING * CHUNK - 1)) // (RING * CHUNK)

    trash = jnp.int32(SW) + s * 4

    for p in range(PASSES):
        lo = c * (NODES_PER_CORE * K) + jnp.int32(p * NP_ * K)

        zdescs = [
            pltpu.async_copy(
                zbuf, slab.at[pl.ds(s * TSHARE + z * ZBUF, ZBUF)], sem)
            for z in range(NZ)
        ]
        for dsc in zdescs:
            dsc.wait()
        plsc.subcore_barrier()

        def scatter_group(g, carry):
            descs = []
            for b in range(RING):
                base = g * (RING * CHUNK) + b * CHUNK
                for v in range(CHUNK // 16):
                    kv = keys[pl.ds(base + v * 16, 16)]
                    off = kv - lo
                    valid = (off >= 0) & (off < SW)
                    idxb[b, pl.ds(v * 16, 16)] = jnp.where(valid, off, trash)
                descs.append(
                    pltpu.async_copy(ones, slab.at[idxb.at[b]], sem, add=True))
            for d in descs:
                d.wait()
            return carry
        lax.fori_loop(0, ngr, scatter_group, 0)
        plsc.subcore_barrier()

        nt_rows = jnp.where(s < 13, 8, 7)
        t0 = jnp.where(s < 13, 8 * s, 104 + (s - 13) * 7)
        pass_base = c * NODES_PER_CORE + jnp.int32(p * NP_)

        def copy_group(gi, carry):
            node_l = (t0 + gi) * 8
            descs = [
                pltpu.async_copy(
                    slab.at[pl.ds((node_l + i) * K, K)], bounce.at[i], sem)
                for i in range(8)
            ]
            for d in descs:
                d.wait()
            pltpu.async_copy(
                bounce, hist_hbm.at[pl.ds(pass_base + node_l, 8), :],
                sem2).wait()
            return carry
        lax.fori_loop(0, nt_rows, copy_group, 0)
        plsc.subcore_barrier()


@jax.jit
def _sc_hist(edge_index, rel, xflat):
    mesh = plsc.VectorSubcoreMesh(core_axis_name="c", subcore_axis_name="s")
    return pl.kernel(
        _sc_body,
        out_type=jax.ShapeDtypeStruct((N, K), jnp.float32),
        mesh=mesh,
        compiler_params=pltpu.CompilerParams(needs_layout_passes=False),
        scratch_types=[
            pltpu.VMEM_SHARED((SLAB,), jnp.float32),
            pltpu.VMEM((N,), jnp.int32),
            pltpu.VMEM((CLCAP,), jnp.int32),
            pltpu.VMEM((STAGE,), jnp.int32),
            pltpu.VMEM((STAGE,), jnp.int32),
            pltpu.VMEM((STAGE,), jnp.int32),
            pltpu.VMEM((RING, CHUNK), jnp.int32),
            pltpu.VMEM((CHUNK,), jnp.float32),
            pltpu.VMEM((ZBUF,), jnp.float32),
            pltpu.VMEM((8, K), jnp.float32),
            pltpu.SemaphoreType.DMA,
            pltpu.SemaphoreType.DMA,
        ],
    )(edge_index, rel, xflat)


BN = 2000
GRID = N // BN
TS = (R + 1) * CP
KP = 1536


def _tc_body(hist_ref, nep_ref, wrel_ref, wroot_ref, bias_ref, x_ref,
             nt_ref, evw_ref, evb_ref, out_ref, ts_ref, big_ref, acc_ref,
             cnt_ref):
    step = pl.program_id(0)

    @pl.when(step == 0)
    def _():
        nep = jnp.concatenate(
            [nep_ref[...], jnp.zeros((CP - C, H), jnp.float32)], axis=0)
        for r in range(R):
            ts_ref[r * CP:(r + 1) * CP, :] = jnp.dot(
                nep, wrel_ref[r], preferred_element_type=jnp.float32)
        ts_ref[R * CP:TS, :] = jnp.dot(
            nep, wroot_ref[...], preferred_element_type=jnp.float32)
        acc_ref[...] = jnp.zeros((1, H), jnp.float32)
        cnt_ref[0, 0] = 0.0

    hist = hist_ref[...]
    for r in range(R):
        hr = hist[:, r * CP:(r + 1) * CP]
        cnt = jnp.sum(hr, axis=1, keepdims=True)
        big_ref[:, r * CP:(r + 1) * CP] = hr * (1.0 / jnp.maximum(cnt, 1.0))
    xb = x_ref[...]
    iot = lax.broadcasted_iota(jnp.int32, (BN, CP), 1)
    oh = (iot == xb).astype(jnp.float32)
    root = jnp.dot(oh, ts_ref[R * CP:TS, :],
                   preferred_element_type=jnp.float32)
    agg = jnp.dot(big_ref[...], ts_ref[:K, :],
                  preferred_element_type=jnp.float32)
    h = jnp.maximum(agg + root + bias_ref[...].reshape(1, H), 0.0)
    m = (nt_ref[...] == 0).astype(jnp.float32)
    acc_ref[...] += jnp.sum(h * m, axis=0, keepdims=True)
    cnt_ref[0, 0] += jnp.sum(m)

    @pl.when(step == pl.num_programs(0) - 1)
    def _():
        g = acc_ref[...] / jnp.maximum(cnt_ref[0, 0], 1.0)
        out_ref[...] = lax.dot_general(
            g, evw_ref[...], (((1,), (1,)), ((), ())),
            preferred_element_type=jnp.float32) + evb_ref[...].reshape(1, NEV)


@jax.jit
def _tc_head(hist2d, nep, W_rel, W_root, bias2, xi, nti, ev_W, evb2):
    return pl.pallas_call(
        _tc_body,
        grid=(GRID,),
        in_specs=[
            pl.BlockSpec((BN, K), lambda i: (i, 0)),
            pl.BlockSpec((C, H), lambda i: (0, 0)),
            pl.BlockSpec((R, H, H), lambda i: (0, 0, 0)),
            pl.BlockSpec((H, H), lambda i: (0, 0)),
            pl.BlockSpec((H,), lambda i: (0,)),
            pl.BlockSpec((BN, 1), lambda i: (i, 0)),
            pl.BlockSpec((BN, 1), lambda i: (i, 0)),
            pl.BlockSpec((NEV, H), lambda i: (0, 0)),
            pl.BlockSpec((NEV,), lambda i: (0,)),
        ],
        out_specs=pl.BlockSpec((1, NEV), lambda i: (0, 0)),
        out_shape=jax.ShapeDtypeStruct((1, NEV), jnp.float32),
        scratch_shapes=[
            pltpu.VMEM((TS, H), jnp.float32),
            pltpu.VMEM((BN, K), jnp.float32),
            pltpu.VMEM((1, H), jnp.float32),
            pltpu.SMEM((1, 1), jnp.float32),
        ],
    )(hist2d, nep, W_rel, W_root, bias2, xi, nti, ev_W, evb2)


def kernel(x, edge_index, edge_type, node_type, node_emb, W_rel, W_root,
           bias, ev_W, ev_b):
    xflat = x[:, 0].astype(jnp.int32)
    hist = _sc_hist(edge_index.astype(jnp.int32).reshape(2 * E),
                    edge_type.astype(jnp.int32), xflat)

    return _tc_head(hist, node_emb, W_rel, W_root, bias, x.astype(jnp.int32),
                    node_type.reshape(N, 1).astype(jnp.int32), ev_W, ev_b)


# speedup vs baseline: 1.2139x; 1.0002x over previous
import jax
import jax.numpy as jnp
from jax import lax
from jax.experimental import pallas as pl
from jax.experimental.pallas import tpu as pltpu
from jax.experimental.pallas import tpu_sc as plsc

N = 10000
E = 320000
R = 8
C = 150
CP = 160
K = R * CP
H = 128
NEV = 49

NC = 2
NS = 16
ET = E // NS
STAGE = 2000
NSTAGE = ET // STAGE
VPC = STAGE // 16
NODES_PER_CORE = N // NC
PASSES = 5
NP_ = NODES_PER_CORE // PASSES
SW = NP_ * K
SLAB = SW + 128
TSHARE = SW // NS
ZBUF = 2000
NZ = TSHARE // ZBUF
CHUNK = 128
RING = 8
CLCAP = ET + 1152


def _sc_body(edge_hbm, rel_hbm, x_hbm, hist_hbm,
             slab, xbuf, keys, srcb, dstb, relb, idxb, ones, zbuf, bounce,
             sem, sem2):
    c = lax.axis_index("c")
    s = lax.axis_index("s")
    tile_base = s * ET

    pltpu.sync_copy(x_hbm, xbuf)

    def fill_ones(i, carry):
        ones[pl.ds(i * 16, 16)] = jnp.ones((16,), jnp.float32)
        return carry
    lax.fori_loop(0, 8, fill_ones, 0)

    def fill_zeros(i, carry):
        zbuf[pl.ds(i * 16, 16)] = jnp.zeros((16,), jnp.float32)
        return carry
    lax.fori_loop(0, ZBUF // 16, fill_zeros, 0)

    lo_core = c * NODES_PER_CORE

    def stage_chunk(j, pos):
        off = tile_base + j * STAGE
        pltpu.sync_copy(edge_hbm.at[pl.ds(off, STAGE)], srcb)
        pltpu.sync_copy(edge_hbm.at[pl.ds(E + off, STAGE)], dstb)
        pltpu.sync_copy(rel_hbm.at[pl.ds(off, STAGE)], relb)

        def keyvec(v, pos2):
            sv = srcb[pl.ds(v * 16, 16)]
            dv = dstb[pl.ds(v * 16, 16)]
            rv = relb[pl.ds(v * 16, 16)]
            cls = plsc.load_gather(xbuf, [sv])
            kv = dv * K + rv * CP + cls
            valid = (dv >= lo_core) & (dv < lo_core + NODES_PER_CORE)
            plsc.store_compressed(keys.at[pl.ds(pos2, 16)], kv, mask=valid)
            return pos2 + jnp.sum(valid.astype(jnp.int32))
        return lax.fori_loop(0, VPC, keyvec, pos)
    pos = lax.fori_loop(0, NSTAGE, stage_chunk, jnp.int32(0))

    def fill_tail(t, carry):
        keys[pl.ds(pos + t * 16, 16)] = jnp.full((16,), -1, jnp.int32)
        return carry
    lax.fori_loop(0, 66, fill_tail, 0)
    ngr = (pos + jnp.int32(RING * CHUNK - 1)) // (RING * CHUNK)

    trash = jnp.int32(SW) + s * 4

    for p in range(PASSES):
        lo = c * (NODES_PER_CORE * K) + jnp.int32(p * NP_ * K)

        zdescs = [
            pltpu.async_copy(
                zbuf, slab.at[pl.ds(s * TSHARE + z * ZBUF, ZBUF)], sem)
            for z in range(NZ)
        ]
        for dsc in zdescs:
            dsc.wait()
        plsc.subcore_barrier()

        def scatter_group(g, carry):
            descs = []
            for b in range(RING):
                base = g * (RING * CHUNK) + b * CHUNK
                for v in range(CHUNK // 16):
                    kv = keys[pl.ds(base + v * 16, 16)]
                    off = kv - lo
                    valid = (off >= 0) & (off < SW)
                    idxb[b, pl.ds(v * 16, 16)] = jnp.where(valid, off, trash)
                descs.append(
                    pltpu.async_copy(ones, slab.at[idxb.at[b]], sem, add=True))
            for d in descs:
                d.wait()
            return carry
        lax.fori_loop(0, ngr, scatter_group, 0)
        plsc.subcore_barrier()

        nt_rows = jnp.where(s < 13, 8, 7)
        t0 = jnp.where(s < 13, 8 * s, 104 + (s - 13) * 7)
        pass_base = c * NODES_PER_CORE + jnp.int32(p * NP_)

        def copy_group(gi, carry):
            node_l = (t0 + gi) * 8
            descs = [
                pltpu.async_copy(
                    slab.at[pl.ds((node_l + i) * K, K)], bounce.at[i], sem)
                for i in range(8)
            ]
            for d in descs:
                d.wait()
            pltpu.async_copy(
                bounce, hist_hbm.at[pl.ds(pass_base + node_l, 8), :],
                sem2).wait()
            return carry
        lax.fori_loop(0, nt_rows, copy_group, 0)
        plsc.subcore_barrier()


@jax.jit
def _sc_hist(edge_index, rel, xflat):
    mesh = plsc.VectorSubcoreMesh(core_axis_name="c", subcore_axis_name="s")
    return pl.kernel(
        _sc_body,
        out_type=jax.ShapeDtypeStruct((N, K), jnp.float32),
        mesh=mesh,
        compiler_params=pltpu.CompilerParams(needs_layout_passes=False),
        scratch_types=[
            pltpu.VMEM_SHARED((SLAB,), jnp.float32),
            pltpu.VMEM((N,), jnp.int32),
            pltpu.VMEM((CLCAP,), jnp.int32),
            pltpu.VMEM((STAGE,), jnp.int32),
            pltpu.VMEM((STAGE,), jnp.int32),
            pltpu.VMEM((STAGE,), jnp.int32),
            pltpu.VMEM((RING, CHUNK), jnp.int32),
            pltpu.VMEM((CHUNK,), jnp.float32),
            pltpu.VMEM((ZBUF,), jnp.float32),
            pltpu.VMEM((8, K), jnp.float32),
            pltpu.SemaphoreType.DMA,
            pltpu.SemaphoreType.DMA,
        ],
    )(edge_index, rel, xflat)


BN = 2000
GRID = N // BN
TS = (R + 1) * CP


def _tc_body(hist_ref, nep_ref, wrel_ref, wroot_ref, bias_ref, x_ref,
             nt_ref, evw_ref, evb_ref, out_ref, ts_ref, big_ref, acc_ref,
             cnt_ref):
    step = pl.program_id(0)

    @pl.when(step == 0)
    def _():
        nep = jnp.concatenate(
            [nep_ref[...], jnp.zeros((CP - C, H), jnp.float32)], axis=0)
        for r in range(R):
            ts_ref[r * CP:(r + 1) * CP, :] = jnp.dot(
                nep, wrel_ref[r], preferred_element_type=jnp.float32)
        ts_ref[R * CP:TS, :] = jnp.dot(
            nep, wroot_ref[...], preferred_element_type=jnp.float32)
        acc_ref[...] = jnp.zeros((1, H), jnp.float32)
        cnt_ref[0, 0] = 0.0

    hist = hist_ref[...]
    for r in range(R):
        hr = hist[:, r * CP:(r + 1) * CP]
        cnt = jnp.sum(hr, axis=1, keepdims=True)
        big_ref[:, r * CP:(r + 1) * CP] = hr * (1.0 / jnp.maximum(cnt, 1.0))
    xb = x_ref[...]
    iot = lax.broadcasted_iota(jnp.int32, (BN, CP), 1)
    oh = (iot == xb).astype(jnp.float32)
    root = jnp.dot(oh, ts_ref[R * CP:TS, :],
                   preferred_element_type=jnp.float32)
    agg = jnp.dot(big_ref[...], ts_ref[:K, :],
                  preferred_element_type=jnp.float32)
    h = jnp.maximum(agg + root + bias_ref[...].reshape(1, H), 0.0)
    m = (nt_ref[...] == 0).astype(jnp.float32)
    acc_ref[...] += jnp.sum(h * m, axis=0, keepdims=True)
    cnt_ref[0, 0] += jnp.sum(m)

    @pl.when(step == pl.num_programs(0) - 1)
    def _():
        g = acc_ref[...] / jnp.maximum(cnt_ref[0, 0], 1.0)
        out_ref[...] = lax.dot_general(
            g, evw_ref[...], (((1,), (1,)), ((), ())),
            preferred_element_type=jnp.float32) + evb_ref[...].reshape(1, NEV)


@jax.jit
def _tc_head(hist2d, nep, W_rel, W_root, bias2, xi, nti, ev_W, evb2):
    return pl.pallas_call(
        _tc_body,
        grid=(GRID,),
        in_specs=[
            pl.BlockSpec((BN, K), lambda i: (i, 0)),
            pl.BlockSpec((C, H), lambda i: (0, 0)),
            pl.BlockSpec((R, H, H), lambda i: (0, 0, 0)),
            pl.BlockSpec((H, H), lambda i: (0, 0)),
            pl.BlockSpec((H,), lambda i: (0,)),
            pl.BlockSpec((BN, 1), lambda i: (i, 0)),
            pl.BlockSpec((BN, 1), lambda i: (i, 0)),
            pl.BlockSpec((NEV, H), lambda i: (0, 0)),
            pl.BlockSpec((NEV,), lambda i: (0,)),
        ],
        out_specs=pl.BlockSpec((1, NEV), lambda i: (0, 0)),
        out_shape=jax.ShapeDtypeStruct((1, NEV), jnp.float32),
        scratch_shapes=[
            pltpu.VMEM((TS, H), jnp.float32),
            pltpu.VMEM((BN, K), jnp.float32),
            pltpu.VMEM((1, H), jnp.float32),
            pltpu.SMEM((1, 1), jnp.float32),
        ],
    )(hist2d, nep, W_rel, W_root, bias2, xi, nti, ev_W, evb2)


def kernel(x, edge_index, edge_type, node_type, node_emb, W_rel, W_root,
           bias, ev_W, ev_b):
    xflat = x[:, 0].astype(jnp.int32)
    hist = _sc_hist(edge_index.astype(jnp.int32).reshape(2 * E),
                    edge_type.astype(jnp.int32), xflat)

    return _tc_head(hist, node_emb, W_rel, W_root, bias, x.astype(jnp.int32),
                    node_type.reshape(N, 1).astype(jnp.int32), ev_W, ev_b)
